# baseline (speedup 1.0000x reference)
import jax
import jax.numpy as jnp
from jax.experimental import pallas as pl

_NUM_BINS = 100000
_SALT_ADD = (42 * 0x9E3779B9) & 0xFFFFFFFF


def _hash_body(x_ref, o_ref):
    z = x_ref[...].astype(jnp.uint32)
    z = z + jnp.uint32(_SALT_ADD)
    z = (z ^ (z >> 16)) * jnp.uint32(0x85EBCA6B)
    z = (z ^ (z >> 13)) * jnp.uint32(0xC2B2AE35)
    z = z ^ (z >> 16)
    q = z // jnp.uint32(_NUM_BINS)
    r = z - q * jnp.uint32(_NUM_BINS)
    o_ref[...] = r.astype(jnp.int32)


def kernel(inputs):
    n, m = inputs.shape
    total = n * m
    cols = 512
    rows = total // cols
    x = inputs.reshape(rows, cols)

    grid = 8
    br = rows // grid
    out = pl.pallas_call(
        _hash_body,
        grid=(grid,),
        in_specs=[pl.BlockSpec((br, cols), lambda i: (i, 0))],
        out_specs=pl.BlockSpec((br, cols), lambda i: (i, 0)),
        out_shape=jax.ShapeDtypeStruct((rows, cols), jnp.int32),
    )(x)
    return out.reshape(n, m)


# speedup vs baseline: 5.8974x; 5.8974x over previous
import jax
import jax.numpy as jnp
from jax.experimental import pallas as pl

_NUM_BINS = 100000
_SALT_ADD = (42 * 0x9E3779B9) & 0xFFFFFFFF


def _hash_body(x_ref, o_ref):
    z = x_ref[...].astype(jnp.uint32)
    z = z + jnp.uint32(_SALT_ADD)
    z = (z ^ (z >> 16)) * jnp.uint32(0x85EBCA6B)
    z = (z ^ (z >> 13)) * jnp.uint32(0xC2B2AE35)
    z = z ^ (z >> 16)
    q = z // jnp.uint32(_NUM_BINS)
    r = z - q * jnp.uint32(_NUM_BINS)
    o_ref[...] = r.astype(jnp.int32)


def kernel(inputs):
    n, m = inputs.shape
    xt = jnp.swapaxes(inputs, 0, 1)
    grid = 16
    bc = n // grid
    out_t = pl.pallas_call(
        _hash_body,
        grid=(grid,),
        in_specs=[pl.BlockSpec((m, bc), lambda i: (0, i))],
        out_specs=pl.BlockSpec((m, bc), lambda i: (0, i)),
        out_shape=jax.ShapeDtypeStruct((m, n), jnp.int32),
    )(xt)
    return jnp.swapaxes(out_t, 0, 1)
